# baseline (speedup 1.0000x reference)
import dataclasses
import functools

import jax
import jax.numpy as jnp
from jax.experimental import pallas as pl
from jax.experimental.pallas import tpu as pltpu
from jax.experimental.pallas import tpu_sc as plsc

_NC = 2
_NS = 16
_NW = _NC * _NS
_LANES = 16

_CHUNK = 8
_NBUF = 2
_SPM_CHUNK = 4
_N_SPM = 8


def _gather_rows(x, valid_idx, n_rows, d):
    per_w = n_rows // _NW
    n_chunks = per_w // _CHUNK
    mesh = plsc.VectorSubcoreMesh(core_axis_name="core",
                                  subcore_axis_name="subcore")
    cp = pltpu.CompilerParams()
    if "needs_layout_passes" in pltpu.CompilerParams.__dataclass_fields__:
        cp = dataclasses.replace(cp, needs_layout_passes=False)

    @functools.partial(
        pl.kernel,
        out_type=jax.ShapeDtypeStruct((n_rows, d), x.dtype),
        mesh=mesh,
        compiler_params=cp,
        scratch_types=[
            pltpu.VMEM((per_w,), jnp.int32),
            pltpu.VMEM((_NBUF, _CHUNK, d), x.dtype),
            pltpu.VMEM_SHARED((_NS, 2, _SPM_CHUNK, d), x.dtype),
            pltpu.SemaphoreType.DMA,
            pltpu.SemaphoreType.DMA,
            pltpu.SemaphoreType.DMA,
            pltpu.SemaphoreType.DMA,
        ],
    )
    def gather_kernel(x_hbm, i_hbm, o_hbm, idx_v, buf, shared,
                      sem_in, sem_out, sem_shin, sem_shout):
        sid = jax.lax.axis_index("subcore")
        wid = jax.lax.axis_index("subcore") * _NC + jax.lax.axis_index("core")
        base = wid * per_w
        pltpu.sync_copy(i_hbm.at[pl.ds(base, per_w)], idx_v)

        lanes = jax.lax.iota(jnp.int32, _LANES)
        contig = None
        for k in range(per_w // _LANES):
            v = idx_v[pl.ds(k * _LANES, _LANES)]
            ok = jnp.all(v == base + k * _LANES + lanes)
            contig = ok if contig is None else jnp.logical_and(contig, ok)

        def ring_pipeline(gather):
            def writeback(c):
                return pltpu.make_async_copy(
                    buf.at[c % _NBUF],
                    o_hbm.at[pl.ds(base + c * _CHUNK, _CHUNK)], sem_out)

            for c in range(min(_NBUF - 1, n_chunks)):
                gather(c).start()
            pending_wb = 0
            for c in range(n_chunks):
                gather(c).wait()
                writeback(c).start()
                pending_wb += 1
                nxt = c + _NBUF - 1
                if nxt < n_chunks:
                    if nxt >= _NBUF:
                        writeback(nxt - _NBUF).wait()
                        pending_wb -= 1
                    gather(nxt).start()
            for _ in range(pending_wb):
                writeback(n_chunks - 1).wait()

        @pl.when(contig)
        def _fast():
            spm_total = _N_SPM * _SPM_CHUNK
            n_str = (per_w - spm_total) // _CHUNK
            spm_rows = base + n_str * _CHUNK

            def g_str(c):
                return pltpu.make_async_copy(
                    x_hbm.at[pl.ds(base + c * _CHUNK, _CHUNK)],
                    buf.at[c % _NBUF], sem_in)

            def w_str(c):
                return pltpu.make_async_copy(
                    buf.at[c % _NBUF],
                    o_hbm.at[pl.ds(base + c * _CHUNK, _CHUNK)], sem_out)

            def g_spm(c):
                return pltpu.make_async_copy(
                    x_hbm.at[pl.ds(spm_rows + c * _SPM_CHUNK, _SPM_CHUNK)],
                    shared.at[sid, c % 2], sem_shin)

            def w_spm(c):
                return pltpu.make_async_copy(
                    shared.at[sid, c % 2],
                    o_hbm.at[pl.ds(spm_rows + c * _SPM_CHUNK, _SPM_CHUNK)],
                    sem_shout)

            def prime(n, nbuf, g):
                for c in range(min(nbuf - 1, n)):
                    g(c).start()

            def step(c, n, nbuf, g, w, state):
                if c >= n:
                    return
                g(c).wait()
                w(c).start()
                nxt = c + nbuf - 1
                if nxt < n:
                    if nxt >= nbuf:
                        w(nxt - nbuf).wait()
                        state["waited"] += 1
                    g(nxt).start()

            st_str = {"waited": 0}
            st_spm = {"waited": 0}
            prime(n_str, _NBUF, g_str)
            prime(_N_SPM, 2, g_spm)
            for c in range(max(n_str, _N_SPM)):
                step(c, n_str, _NBUF, g_str, w_str, st_str)
                step(c, _N_SPM, 2, g_spm, w_spm, st_spm)
            for _ in range(n_str - st_str["waited"]):
                w_str(n_str - 1).wait()
            for _ in range(_N_SPM - st_spm["waited"]):
                w_spm(_N_SPM - 1).wait()

        @pl.when(jnp.logical_not(contig))
        def _slow():
            ring_pipeline(lambda c: pltpu.make_async_copy(
                x_hbm.at[idx_v.at[pl.ds(c * _CHUNK, _CHUNK)]],
                buf.at[c % _NBUF], sem_in))

    return gather_kernel(x, valid_idx)


def kernel(x, valid_idx):
    n_rows = valid_idx.shape[0]
    d = x.shape[1]
    return _gather_rows(x, valid_idx, n_rows, d)


# speedup vs baseline: 1.0222x; 1.0222x over previous
import functools

import jax
import jax.numpy as jnp
from jax.experimental import pallas as pl
from jax.experimental.pallas import tpu as pltpu
from jax.experimental.pallas import tpu_sc as plsc

_NC = 2
_NS = 16
_NW = _NC * _NS
_CHUNK = 8
_NBUF = 3


def _gather_rows(x, valid_idx, n_rows, d):
    per_w = n_rows // _NW
    n_chunks = per_w // _CHUNK
    mesh = plsc.VectorSubcoreMesh(core_axis_name="core",
                                  subcore_axis_name="subcore")

    @functools.partial(
        pl.kernel,
        out_type=jax.ShapeDtypeStruct((n_rows, d), x.dtype),
        mesh=mesh,
        scratch_types=[
            pltpu.VMEM((per_w,), jnp.int32),
            pltpu.VMEM((_NBUF, _CHUNK, d), x.dtype),
            pltpu.SemaphoreType.DMA,
            pltpu.SemaphoreType.DMA,
        ],
    )
    def gather_kernel(x_hbm, i_hbm, o_hbm, idx_v, buf, sem_in, sem_out):
        wid = jax.lax.axis_index("subcore") * _NC + jax.lax.axis_index("core")
        base = wid * per_w
        pltpu.sync_copy(i_hbm.at[pl.ds(base, per_w)], idx_v)

        def gather(c):
            return pltpu.make_async_copy(
                x_hbm.at[idx_v.at[pl.ds(c * _CHUNK, _CHUNK)]],
                buf.at[c % _NBUF], sem_in)

        def writeback(c):
            return pltpu.make_async_copy(
                buf.at[c % _NBUF], o_hbm.at[pl.ds(base + c * _CHUNK, _CHUNK)],
                sem_out)

        for c in range(min(_NBUF - 1, n_chunks)):
            gather(c).start()
        pending_wb = 0
        for c in range(n_chunks):
            gather(c).wait()
            writeback(c).start()
            pending_wb += 1
            nxt = c + _NBUF - 1
            if nxt < n_chunks:
                if pending_wb > _NBUF - 2:
                    writeback(c - 1).wait()
                    pending_wb -= 1
                gather(nxt).start()
        for _ in range(pending_wb):
            writeback(n_chunks - 1).wait()

    return gather_kernel(x, valid_idx)


def kernel(x, valid_idx):
    n_rows = valid_idx.shape[0]
    d = x.shape[1]
    return _gather_rows(x, valid_idx, n_rows, d)
